# Initial kernel scaffold; baseline (speedup 1.0000x reference)
import jax
import jax.numpy as jnp
from jax.experimental import pallas as pl


def kernel(H, edge_index, edge_attr, alpha, W_gcn, W_e, W_gine, eps):
    raise NotImplementedError("write your pallas kernel here")


# speedup vs baseline: 5.9959x; 5.9959x over previous
import functools

import jax
import jax.numpy as jnp
from jax import lax
from jax.experimental import pallas as pl
from jax.experimental.pallas import tpu as pltpu
from jax.experimental.pallas import tpu_sc as plsc

N = 10000
E = 320000
D = 128
DE = 16

CHUNK = 128
NCHUNK = E // CHUNK
NTILE = 16
NPAD = 10240
ROWS_PER_TILE = NPAD // NTILE

_MESH = plsc.VectorSubcoreMesh(core_axis_name="c", subcore_axis_name="s")


@functools.partial(
    pl.kernel,
    out_type=jax.ShapeDtypeStruct((2, NPAD, D), jnp.float32),
    mesh=_MESH,
    scratch_types=[
        pltpu.VMEM((CHUNK, D), jnp.float32),
        pltpu.VMEM((1, CHUNK), jnp.int32),
        pltpu.VMEM((CHUNK, D), jnp.float32),
        pltpu.VMEM_SHARED((NPAD, D), jnp.float32),
    ],
)
def _sc_degree(dst_hbm, out_hbm, zb_v, idx_v, ones_v, acc_sh):
    c = lax.axis_index("c")
    s = lax.axis_index("s")

    @pl.loop(0, CHUNK)
    def _(r):
        for k in range(D // 16):
            zb_v[r, pl.ds(16 * k, 16)] = jnp.zeros((16,), jnp.float32)
            ones_v[r, pl.ds(16 * k, 16)] = jnp.ones((16,), jnp.float32)

    for q in range(5):
        pltpu.sync_copy(
            zb_v, acc_sh.at[pl.ds(s * ROWS_PER_TILE + q * CHUNK, CHUNK)])
    plsc.subcore_barrier()

    @pl.loop(0, 79)
    def _(i):
        j = s + 16 * i

        @pl.when(j < NCHUNK // 2)
        def _():
            jj = c * (NCHUNK // 2) + j
            pltpu.sync_copy(dst_hbm.at[jj], idx_v)
            pltpu.sync_copy(ones_v, acc_sh.at[idx_v.at[0]], add=True)

    plsc.subcore_barrier()
    for q in range(5):
        pltpu.sync_copy(
            acc_sh.at[pl.ds(s * ROWS_PER_TILE + q * CHUNK, CHUNK)],
            out_hbm.at[c].at[pl.ds(s * ROWS_PER_TILE + q * CHUNK, CHUNK)],
        )


@functools.partial(
    pl.kernel,
    out_type=jax.ShapeDtypeStruct((2, NPAD, D), jnp.float32),
    mesh=_MESH,
    scratch_types=[
        pltpu.VMEM((1, CHUNK), jnp.int32),
        pltpu.VMEM((1, CHUNK), jnp.int32),
        pltpu.VMEM((CHUNK, D), jnp.float32),
        pltpu.VMEM((CHUNK, D), jnp.float32),
        pltpu.VMEM_SHARED((NPAD, D), jnp.float32),
        pltpu.SemaphoreType.DMA,
    ],
)
def _sc_main(xw2_hbm, h_hbm, e_hbm, src_hbm, dst_hbm, out_hbm,
             srcv, dstv, rows, ech, acc_sh, sem):
    c = lax.axis_index("c")
    s = lax.axis_index("s")

    @pl.loop(0, CHUNK)
    def _(r):
        for k in range(D // 16):
            rows[r, pl.ds(16 * k, 16)] = jnp.zeros((16,), jnp.float32)

    for q in range(5):
        pltpu.sync_copy(
            rows, acc_sh.at[pl.ds(s * ROWS_PER_TILE + q * CHUNK, CHUNK)])
    plsc.subcore_barrier()

    @pl.when(c == 0)
    def _():
        @pl.loop(0, 157)
        def _(i):
            j = s + 16 * i

            @pl.when(j < NCHUNK)
            def _():
                pltpu.sync_copy(src_hbm.at[j], srcv)
                pltpu.sync_copy(dst_hbm.at[j], dstv)
                pltpu.async_copy(xw2_hbm.at[srcv.at[0]], rows, sem).wait()
                pltpu.sync_copy(rows, acc_sh.at[dstv.at[0]], add=True)

    @pl.when(c == 1)
    def _():
        @pl.loop(0, 157)
        def _(i):
            j = s + 16 * i

            @pl.when(j < NCHUNK)
            def _():
                pltpu.sync_copy(src_hbm.at[j], srcv)
                pltpu.sync_copy(dst_hbm.at[j], dstv)
                pltpu.async_copy(h_hbm.at[srcv.at[0]], rows, sem).wait()
                pltpu.sync_copy(e_hbm.at[pl.ds(j * CHUNK, CHUNK)], ech)

                @pl.loop(0, CHUNK)
                def _(r):
                    for k in range(D // 16):
                        a = rows[r, pl.ds(16 * k, 16)]
                        b = ech[r, pl.ds(16 * k, 16)]
                        rows[r, pl.ds(16 * k, 16)] = jnp.maximum(a + b, 0.0)

                pltpu.sync_copy(rows, acc_sh.at[dstv.at[0]], add=True)

    plsc.subcore_barrier()
    for q in range(5):
        pltpu.sync_copy(
            acc_sh.at[pl.ds(s * ROWS_PER_TILE + q * CHUNK, CHUNK)],
            out_hbm.at[c].at[pl.ds(s * ROWS_PER_TILE + q * CHUNK, CHUNK)],
        )


def _softmax_w(alpha_ref):
    al = alpha_ref[...]
    ex = jnp.exp(al - jnp.max(al))
    return ex / jnp.sum(ex)


def _tc_prep_body(h_ref, w_ref, degp_ref, alpha_ref, xw2_ref, dis_ref):
    w = _softmax_w(alpha_ref)
    w0 = w[0:1, 0:1]
    deg = degp_ref[0, :, 0:1] + degp_ref[1, :, 0:1]
    dis = jnp.where(deg > 0, lax.rsqrt(jnp.maximum(deg, 1.0)), 0.0)
    xw = jnp.dot(h_ref[...], w_ref[...], preferred_element_type=jnp.float32,
                 precision=lax.Precision.HIGHEST)
    xw2_ref[...] = xw * (dis * w0)
    dis_ref[...] = dis


def _tc_prep(H, W_gcn, degp, alpha2):
    B = 2000
    return pl.pallas_call(
        _tc_prep_body,
        grid=(N // B,),
        in_specs=[
            pl.BlockSpec((B, D), lambda i: (i, 0)),
            pl.BlockSpec((D, D), lambda i: (0, 0)),
            pl.BlockSpec((2, B, D), lambda i: (0, i, 0)),
            pl.BlockSpec((1, 2), lambda i: (0, 0)),
        ],
        out_specs=[
            pl.BlockSpec((B, D), lambda i: (i, 0)),
            pl.BlockSpec((B, 1), lambda i: (i, 0)),
        ],
        out_shape=[
            jax.ShapeDtypeStruct((N, D), jnp.float32),
            jax.ShapeDtypeStruct((N, 1), jnp.float32),
        ],
    )(H, W_gcn, degp, alpha2)


def _tc_etab_body(ea_ref, we_ref, e_ref):
    e_ref[...] = jnp.dot(ea_ref[...], we_ref[...],
                         preferred_element_type=jnp.float32,
                         precision=lax.Precision.HIGHEST)


def _tc_etab(edge_attr, W_e):
    B = 3200
    return pl.pallas_call(
        _tc_etab_body,
        grid=(E // B,),
        in_specs=[
            pl.BlockSpec((B, DE), lambda i: (i, 0)),
            pl.BlockSpec((DE, D), lambda i: (0, 0)),
        ],
        out_specs=pl.BlockSpec((B, D), lambda i: (i, 0)),
        out_shape=jax.ShapeDtypeStruct((E, D), jnp.float32),
    )(edge_attr, W_e)


def _tc_final_body(h_ref, sc_ref, dis_ref, wg_ref, alpha_ref, eps_ref, o_ref):
    w = _softmax_w(alpha_ref)
    w1 = w[0:1, 1:2]
    h = h_ref[...]
    pre = h * (1.0 + eps_ref[...]) + sc_ref[1]
    gine = jnp.dot(pre, wg_ref[...], preferred_element_type=jnp.float32,
                   precision=lax.Precision.HIGHEST)
    o_ref[...] = dis_ref[...] * sc_ref[0] + gine * w1


def _tc_final(H, scout, dis, W_gine, alpha2, eps2):
    B = 2000
    return pl.pallas_call(
        _tc_final_body,
        grid=(N // B,),
        in_specs=[
            pl.BlockSpec((B, D), lambda i: (i, 0)),
            pl.BlockSpec((2, B, D), lambda i: (0, i, 0)),
            pl.BlockSpec((B, 1), lambda i: (i, 0)),
            pl.BlockSpec((D, D), lambda i: (0, 0)),
            pl.BlockSpec((1, 2), lambda i: (0, 0)),
            pl.BlockSpec((1, 1), lambda i: (0, 0)),
        ],
        out_specs=pl.BlockSpec((B, D), lambda i: (i, 0)),
        out_shape=jax.ShapeDtypeStruct((N, D), jnp.float32),
    )(H, scout, dis, W_gine, alpha2, eps2)


def kernel(H, edge_index, edge_attr, alpha, W_gcn, W_e, W_gine, eps):
    src2d = edge_index[0].reshape(NCHUNK, 1, CHUNK)
    dst2d = edge_index[1].reshape(NCHUNK, 1, CHUNK)
    alpha2 = alpha.reshape(1, 2)
    eps2 = eps.reshape(1, 1)

    degp = _sc_degree(dst2d)
    etab = _tc_etab(edge_attr, W_e)
    xw2, dis = _tc_prep(H, W_gcn, degp, alpha2)
    scout = _sc_main(xw2, H, etab, src2d, dst2d)
    return _tc_final(H, scout, dis, W_gine, alpha2, eps2)


# speedup vs baseline: 6.8138x; 1.1364x over previous
import functools

import jax
import jax.numpy as jnp
from jax import lax
from jax.experimental import pallas as pl
from jax.experimental.pallas import tpu as pltpu
from jax.experimental.pallas import tpu_sc as plsc

N = 10000
E = 320000
D = 128
DE = 16

CHUNK = 128
NCHUNK = E // CHUNK
NCHUNK_PAD = 2512
NTILE = 16
NPAD = 10240
ROWS_PER_TILE = NPAD // NTILE

RING = 16
NWIN = 10
EQ = 32
DEG_WIN = 79

_MESH = plsc.VectorSubcoreMesh(core_axis_name="c", subcore_axis_name="s")


@functools.partial(
    pl.kernel,
    out_type=jax.ShapeDtypeStruct((2, NPAD, D), jnp.float32),
    mesh=_MESH,
    scratch_types=[
        pltpu.VMEM((CHUNK, D), jnp.float32),
        pltpu.VMEM((DEG_WIN, 1, CHUNK), jnp.int32),
        pltpu.VMEM((CHUNK, D), jnp.float32),
        pltpu.VMEM_SHARED((NPAD, D), jnp.float32),
    ],
)
def _sc_degree(dst_hbm, out_hbm, zb_v, idx_v, ones_v, acc_sh):
    c = lax.axis_index("c")
    s = lax.axis_index("s")

    @pl.loop(0, CHUNK)
    def _(r):
        for k in range(D // 16):
            zb_v[r, pl.ds(16 * k, 16)] = jnp.zeros((16,), jnp.float32)
            ones_v[r, pl.ds(16 * k, 16)] = jnp.ones((16,), jnp.float32)

    start = c * (NCHUNK // 2) + s * 78 + jnp.minimum(s, 2)
    cnt = jnp.where(s < 2, 79, 78)
    pltpu.sync_copy(dst_hbm.at[pl.ds(start, DEG_WIN)], idx_v)

    for q in range(5):
        pltpu.sync_copy(
            zb_v, acc_sh.at[pl.ds(s * ROWS_PER_TILE + q * CHUNK, CHUNK)])
    plsc.subcore_barrier()

    @pl.loop(0, DEG_WIN)
    def _(i):
        @pl.when(i < cnt)
        def _():
            pltpu.sync_copy(ones_v, acc_sh.at[idx_v.at[i].at[0]], add=True)

    plsc.subcore_barrier()
    for q in range(5):
        pltpu.sync_copy(
            acc_sh.at[pl.ds(s * ROWS_PER_TILE + q * CHUNK, CHUNK)],
            out_hbm.at[c].at[pl.ds(s * ROWS_PER_TILE + q * CHUNK, CHUNK)],
        )


@functools.partial(
    pl.kernel,
    out_type=jax.ShapeDtypeStruct((2, NPAD, D), jnp.float32),
    mesh=_MESH,
    scratch_types=[
        pltpu.VMEM((RING, 1, CHUNK), jnp.int32),
        pltpu.VMEM((RING, 1, CHUNK), jnp.int32),
        pltpu.VMEM((CHUNK, D), jnp.float32),
        pltpu.VMEM((CHUNK, D), jnp.float32),
        pltpu.VMEM((EQ, D), jnp.float32),
        pltpu.VMEM((EQ, D), jnp.float32),
        pltpu.VMEM_SHARED((NPAD, D), jnp.float32),
        pltpu.SemaphoreType.DMA,
        pltpu.SemaphoreType.DMA,
        pltpu.SemaphoreType.DMA,
        pltpu.SemaphoreType.DMA,
    ],
)
def _sc_main(xw2_hbm, h_hbm, e_hbm, src_hbm, dst_hbm, out_hbm,
             srcv, dstv, rows0, rows1, eq0, eq1, acc_sh,
             semg0, semg1, seme0, seme1):
    c = lax.axis_index("c")
    s = lax.axis_index("s")
    rows = (rows0, rows1)
    eqb = (eq0, eq1)
    semg = (semg0, semg1)
    seme = (seme0, seme1)

    start = s * 156 + jnp.minimum(s, 4)
    cnt = jnp.where(s < 4, 157, 156)

    @pl.loop(0, CHUNK)
    def _(r):
        for k in range(D // 16):
            rows0[r, pl.ds(16 * k, 16)] = jnp.zeros((16,), jnp.float32)

    for q in range(5):
        pltpu.sync_copy(
            rows0, acc_sh.at[pl.ds(s * ROWS_PER_TILE + q * CHUNK, CHUNK)])
    plsc.subcore_barrier()

    tab_hbm = (xw2_hbm, h_hbm)

    def window(w, core):
        pltpu.sync_copy(src_hbm.at[pl.ds(start + w * RING, RING)], srcv)
        pltpu.sync_copy(dst_hbm.at[pl.ds(start + w * RING, RING)], dstv)
        tab = tab_hbm[core]

        for par in (0, 1):
            i = w * RING + par

            @pl.when(i < cnt)
            def _():
                pltpu.make_async_copy(
                    tab.at[srcv.at[par].at[0]], rows[par], semg[par]).start()

        @pl.loop(0, RING // 2)
        def _(t):
            for par in (0, 1):
                r = 2 * t + par
                i = w * RING + r

                @pl.when(i < cnt)
                def _():
                    pltpu.make_async_copy(
                        tab.at[srcv.at[r].at[0]], rows[par], semg[par]).wait()

                    if core == 1:
                        j = start + i
                        pltpu.make_async_copy(
                            e_hbm.at[pl.ds(j * CHUNK, EQ)], eqb[0],
                            seme[0]).start()
                        for q in range(CHUNK // EQ):
                            pltpu.make_async_copy(
                                e_hbm.at[pl.ds(j * CHUNK + q * EQ, EQ)],
                                eqb[q % 2], seme[q % 2]).wait()
                            if q + 1 < CHUNK // EQ:
                                pltpu.make_async_copy(
                                    e_hbm.at[
                                        pl.ds(j * CHUNK + (q + 1) * EQ, EQ)],
                                    eqb[(q + 1) % 2], seme[(q + 1) % 2]
                                ).start()

                            @pl.loop(0, EQ)
                            def _(rr):
                                for k in range(D // 16):
                                    a = rows[par][q * EQ + rr,
                                                  pl.ds(16 * k, 16)]
                                    b = eqb[q % 2][rr, pl.ds(16 * k, 16)]
                                    rows[par][q * EQ + rr,
                                              pl.ds(16 * k, 16)] = (
                                        jnp.maximum(a + b, 0.0))

                    pltpu.sync_copy(
                        rows[par], acc_sh.at[dstv.at[r].at[0]], add=True)

                    @pl.when((r + 2 < RING) & (i + 2 < cnt))
                    def _():
                        pltpu.make_async_copy(
                            tab.at[srcv.at[r + 2].at[0]], rows[par],
                            semg[par]).start()

    @pl.when(c == 0)
    def _():
        @pl.loop(0, NWIN)
        def _(w):
            window(w, 0)

    @pl.when(c == 1)
    def _():
        @pl.loop(0, NWIN)
        def _(w):
            window(w, 1)

    plsc.subcore_barrier()
    for q in range(5):
        pltpu.sync_copy(
            acc_sh.at[pl.ds(s * ROWS_PER_TILE + q * CHUNK, CHUNK)],
            out_hbm.at[c].at[pl.ds(s * ROWS_PER_TILE + q * CHUNK, CHUNK)],
        )


def _softmax_w(alpha_ref):
    al = alpha_ref[...]
    ex = jnp.exp(al - jnp.max(al))
    return ex / jnp.sum(ex)


def _tc_prep_body(h_ref, w_ref, degp_ref, alpha_ref, xw2_ref, dis_ref):
    w = _softmax_w(alpha_ref)
    w0 = w[0:1, 0:1]
    deg = degp_ref[0, :, 0:1] + degp_ref[1, :, 0:1]
    dis = jnp.where(deg > 0, lax.rsqrt(jnp.maximum(deg, 1.0)), 0.0)
    xw = jnp.dot(h_ref[...], w_ref[...], preferred_element_type=jnp.float32,
                 precision=lax.Precision.HIGHEST)
    xw2_ref[...] = xw * (dis * w0)
    dis_ref[...] = dis


def _tc_prep(H, W_gcn, degp, alpha2):
    B = 2000
    return pl.pallas_call(
        _tc_prep_body,
        grid=(N // B,),
        in_specs=[
            pl.BlockSpec((B, D), lambda i: (i, 0)),
            pl.BlockSpec((D, D), lambda i: (0, 0)),
            pl.BlockSpec((2, B, D), lambda i: (0, i, 0)),
            pl.BlockSpec((1, 2), lambda i: (0, 0)),
        ],
        out_specs=[
            pl.BlockSpec((B, D), lambda i: (i, 0)),
            pl.BlockSpec((B, 1), lambda i: (i, 0)),
        ],
        out_shape=[
            jax.ShapeDtypeStruct((N, D), jnp.float32),
            jax.ShapeDtypeStruct((N, 1), jnp.float32),
        ],
    )(H, W_gcn, degp, alpha2)


def _tc_etab_body(ea_ref, we_ref, e_ref):
    e_ref[...] = jnp.dot(ea_ref[...], we_ref[...],
                         preferred_element_type=jnp.float32,
                         precision=lax.Precision.HIGHEST)


def _tc_etab(edge_attr, W_e):
    B = 3200
    return pl.pallas_call(
        _tc_etab_body,
        grid=(E // B,),
        in_specs=[
            pl.BlockSpec((B, DE), lambda i: (i, 0)),
            pl.BlockSpec((DE, D), lambda i: (0, 0)),
        ],
        out_specs=pl.BlockSpec((B, D), lambda i: (i, 0)),
        out_shape=jax.ShapeDtypeStruct((E, D), jnp.float32),
    )(edge_attr, W_e)


def _tc_final_body(h_ref, sc_ref, dis_ref, wg_ref, alpha_ref, eps_ref, o_ref):
    w = _softmax_w(alpha_ref)
    w1 = w[0:1, 1:2]
    h = h_ref[...]
    pre = h * (1.0 + eps_ref[...]) + sc_ref[1]
    gine = jnp.dot(pre, wg_ref[...], preferred_element_type=jnp.float32,
                   precision=lax.Precision.HIGHEST)
    o_ref[...] = dis_ref[...] * sc_ref[0] + gine * w1


def _tc_final(H, scout, dis, W_gine, alpha2, eps2):
    B = 2000
    return pl.pallas_call(
        _tc_final_body,
        grid=(N // B,),
        in_specs=[
            pl.BlockSpec((B, D), lambda i: (i, 0)),
            pl.BlockSpec((2, B, D), lambda i: (0, i, 0)),
            pl.BlockSpec((B, 1), lambda i: (i, 0)),
            pl.BlockSpec((D, D), lambda i: (0, 0)),
            pl.BlockSpec((1, 2), lambda i: (0, 0)),
            pl.BlockSpec((1, 1), lambda i: (0, 0)),
        ],
        out_specs=pl.BlockSpec((B, D), lambda i: (i, 0)),
        out_shape=jax.ShapeDtypeStruct((N, D), jnp.float32),
    )(H, scout, dis, W_gine, alpha2, eps2)


def kernel(H, edge_index, edge_attr, alpha, W_gcn, W_e, W_gine, eps):
    ei_pad = jnp.pad(edge_index, ((0, 0), (0, NCHUNK_PAD * CHUNK - E)))
    src2d = ei_pad[0].reshape(NCHUNK_PAD, 1, CHUNK)
    dst2d = ei_pad[1].reshape(NCHUNK_PAD, 1, CHUNK)
    alpha2 = alpha.reshape(1, 2)
    eps2 = eps.reshape(1, 1)

    degp = _sc_degree(dst2d)
    etab = _tc_etab(edge_attr, W_e)
    xw2, dis = _tc_prep(H, W_gcn, degp, alpha2)
    scout = _sc_main(xw2, H, etab, src2d, dst2d)
    return _tc_final(H, scout, dis, W_gine, alpha2, eps2)


# speedup vs baseline: 9.5476x; 1.4012x over previous
import functools

import jax
import jax.numpy as jnp
from jax import lax
from jax.experimental import pallas as pl
from jax.experimental.pallas import tpu as pltpu
from jax.experimental.pallas import tpu_sc as plsc

N = 10000
E = 320000
D = 128
DE = 16

CHUNK = 128
NCHUNK = E // CHUNK
NCHUNK_PAD = 2512
NTILE = 16
NPAD = 10240
ROWS_PER_TILE = NPAD // NTILE

RING = 16
NWIN = 10
EQ = 32
DEG_WIN = 79

_MESH = plsc.VectorSubcoreMesh(core_axis_name="c", subcore_axis_name="s")


@functools.partial(
    pl.kernel,
    out_type=jax.ShapeDtypeStruct((2, NPAD, D), jnp.float32),
    mesh=_MESH,
    scratch_types=[
        pltpu.VMEM((CHUNK, D), jnp.float32),
        pltpu.VMEM((DEG_WIN, 1, CHUNK), jnp.int32),
        pltpu.VMEM((CHUNK, D), jnp.float32),
        pltpu.VMEM_SHARED((NPAD, D), jnp.float32),
    ],
)
def _sc_degree(dst_hbm, out_hbm, zb_v, idx_v, ones_v, acc_sh):
    c = lax.axis_index("c")
    s = lax.axis_index("s")

    @pl.loop(0, CHUNK)
    def _(r):
        for k in range(D // 16):
            zb_v[r, pl.ds(16 * k, 16)] = jnp.zeros((16,), jnp.float32)
            ones_v[r, pl.ds(16 * k, 16)] = jnp.ones((16,), jnp.float32)

    start = c * (NCHUNK // 2) + s * 78 + jnp.minimum(s, 2)
    cnt = jnp.where(s < 2, 79, 78)
    pltpu.sync_copy(dst_hbm.at[pl.ds(start, DEG_WIN)], idx_v)

    for q in range(5):
        pltpu.sync_copy(
            zb_v, acc_sh.at[pl.ds(s * ROWS_PER_TILE + q * CHUNK, CHUNK)])
    plsc.subcore_barrier()

    @pl.loop(0, DEG_WIN)
    def _(i):
        @pl.when(i < cnt)
        def _():
            pltpu.sync_copy(ones_v, acc_sh.at[idx_v.at[i].at[0]], add=True)

    plsc.subcore_barrier()
    for q in range(5):
        pltpu.sync_copy(
            acc_sh.at[pl.ds(s * ROWS_PER_TILE + q * CHUNK, CHUNK)],
            out_hbm.at[c].at[pl.ds(s * ROWS_PER_TILE + q * CHUNK, CHUNK)],
        )


@functools.partial(
    pl.kernel,
    out_type=jax.ShapeDtypeStruct((2, NPAD, D), jnp.float32),
    mesh=_MESH,
    scratch_types=[
        pltpu.VMEM((RING, 1, CHUNK), jnp.int32),
        pltpu.VMEM((RING, 1, CHUNK), jnp.int32),
        pltpu.VMEM((CHUNK, D), jnp.float32),
        pltpu.VMEM((CHUNK, D), jnp.float32),
        pltpu.VMEM_SHARED((NPAD, D), jnp.float32),
        pltpu.SemaphoreType.DMA,
        pltpu.SemaphoreType.DMA,
        pltpu.SemaphoreType.DMA,
        pltpu.SemaphoreType.DMA,
    ],
)
def _sc_main(xw2_hbm, h_hbm, e_hbm, src_hbm, dst_hbm, out_hbm,
             srcv, dstv, rows0, rows1, acc_sh,
             semg0, semg1, seme0, seme1):
    c = lax.axis_index("c")
    s = lax.axis_index("s")
    rows = (rows0, rows1)
    semg = (semg0, semg1)
    seme = (seme0, seme1)

    start = s * 156 + jnp.minimum(s, 4)
    cnt = jnp.where(s < 4, 157, 156)

    @pl.loop(0, CHUNK)
    def _(r):
        for k in range(D // 16):
            rows0[r, pl.ds(16 * k, 16)] = jnp.zeros((16,), jnp.float32)

    for q in range(5):
        pltpu.sync_copy(
            rows0, acc_sh.at[pl.ds(s * ROWS_PER_TILE + q * CHUNK, CHUNK)])
    plsc.subcore_barrier()

    def gcn_window(w):
        pltpu.sync_copy(src_hbm.at[pl.ds(start + w * RING, RING)], srcv)
        pltpu.sync_copy(dst_hbm.at[pl.ds(start + w * RING, RING)], dstv)

        for par in (0, 1):
            i = w * RING + par

            @pl.when(i < cnt)
            def _():
                pltpu.make_async_copy(
                    xw2_hbm.at[srcv.at[par].at[0]], rows[par],
                    semg[par]).start()

        @pl.loop(0, RING // 2)
        def _(t):
            for par in (0, 1):
                r = 2 * t + par
                i = w * RING + r

                @pl.when(i < cnt)
                def _():
                    pltpu.make_async_copy(
                        xw2_hbm.at[srcv.at[r].at[0]], rows[par],
                        semg[par]).wait()
                    pltpu.sync_copy(
                        rows[par], acc_sh.at[dstv.at[r].at[0]], add=True)

                    @pl.when((r + 2 < RING) & (i + 2 < cnt))
                    def _():
                        pltpu.make_async_copy(
                            xw2_hbm.at[srcv.at[r + 2].at[0]], rows[par],
                            semg[par]).start()

    def gine_window(w):
        pltpu.sync_copy(src_hbm.at[pl.ds(start + w * RING, RING)], srcv)
        pltpu.sync_copy(dst_hbm.at[pl.ds(start + w * RING, RING)], dstv)

        @pl.when(w * RING < cnt)
        def _():
            pltpu.make_async_copy(
                e_hbm.at[pl.ds((start + w * RING) * CHUNK, CHUNK)],
                rows[0], seme[0]).wait()
            pltpu.make_async_copy(
                h_hbm.at[srcv.at[0].at[0]], rows[0], semg[0]
            ).start(add=True)

        @pl.loop(0, RING // 2)
        def _(t):
            for par in (0, 1):
                r = 2 * t + par
                i = w * RING + r

                @pl.when(i < cnt)
                def _():
                    pltpu.make_async_copy(
                        h_hbm.at[srcv.at[r].at[0]], rows[par],
                        semg[par]).wait()

                    @pl.when((r + 1 < RING) & (i + 1 < cnt))
                    def _():
                        pltpu.make_async_copy(
                            e_hbm.at[pl.ds((start + i + 1) * CHUNK, CHUNK)],
                            rows[1 - par], seme[1 - par]).wait()
                        pltpu.make_async_copy(
                            h_hbm.at[srcv.at[r + 1].at[0]], rows[1 - par],
                            semg[1 - par]).start(add=True)

                    @pl.loop(0, CHUNK)
                    def _(rr):
                        for k in range(D // 16):
                            a = rows[par][rr, pl.ds(16 * k, 16)]
                            rows[par][rr, pl.ds(16 * k, 16)] = jnp.maximum(
                                a, 0.0)

                    pltpu.sync_copy(
                        rows[par], acc_sh.at[dstv.at[r].at[0]], add=True)

                    @pl.when(i + 2 < cnt)
                    def _():
                        pltpu.make_async_copy(
                            e_hbm.at[pl.ds((start + i + 2) * CHUNK, CHUNK)],
                            rows[par], seme[par]).start()

    @pl.when(c == 0)
    def _():
        @pl.loop(0, NWIN)
        def _(w):
            gcn_window(w)

    @pl.when(c == 1)
    def _():
        for par in (0, 1):
            pltpu.make_async_copy(
                e_hbm.at[pl.ds((start + par) * CHUNK, CHUNK)], rows[par],
                seme[par]).start()

        @pl.loop(0, NWIN)
        def _(w):
            gine_window(w)

    plsc.subcore_barrier()
    for q in range(5):
        pltpu.sync_copy(
            acc_sh.at[pl.ds(s * ROWS_PER_TILE + q * CHUNK, CHUNK)],
            out_hbm.at[c].at[pl.ds(s * ROWS_PER_TILE + q * CHUNK, CHUNK)],
        )


def _softmax_w(alpha_ref):
    al = alpha_ref[...]
    ex = jnp.exp(al - jnp.max(al))
    return ex / jnp.sum(ex)


def _tc_prep_body(h_ref, w_ref, degp_ref, alpha_ref, xw2_ref, dis_ref):
    w = _softmax_w(alpha_ref)
    w0 = w[0:1, 0:1]
    deg = degp_ref[0, :, 0:1] + degp_ref[1, :, 0:1]
    dis = jnp.where(deg > 0, lax.rsqrt(jnp.maximum(deg, 1.0)), 0.0)
    xw = jnp.dot(h_ref[...], w_ref[...], preferred_element_type=jnp.float32,
                 precision=lax.Precision.HIGHEST)
    xw2_ref[...] = xw * (dis * w0)
    dis_ref[...] = dis


def _tc_prep(H, W_gcn, degp, alpha2):
    B = 2000
    return pl.pallas_call(
        _tc_prep_body,
        grid=(N // B,),
        in_specs=[
            pl.BlockSpec((B, D), lambda i: (i, 0)),
            pl.BlockSpec((D, D), lambda i: (0, 0)),
            pl.BlockSpec((2, B, D), lambda i: (0, i, 0)),
            pl.BlockSpec((1, 2), lambda i: (0, 0)),
        ],
        out_specs=[
            pl.BlockSpec((B, D), lambda i: (i, 0)),
            pl.BlockSpec((B, 1), lambda i: (i, 0)),
        ],
        out_shape=[
            jax.ShapeDtypeStruct((N, D), jnp.float32),
            jax.ShapeDtypeStruct((N, 1), jnp.float32),
        ],
    )(H, W_gcn, degp, alpha2)


def _tc_etab_body(ea_ref, we_ref, e_ref):
    e_ref[...] = jnp.dot(ea_ref[...], we_ref[...],
                         preferred_element_type=jnp.float32,
                         precision=lax.Precision.HIGHEST)


def _tc_etab(edge_attr, W_e):
    B = 3200
    return pl.pallas_call(
        _tc_etab_body,
        grid=(E // B,),
        in_specs=[
            pl.BlockSpec((B, DE), lambda i: (i, 0)),
            pl.BlockSpec((DE, D), lambda i: (0, 0)),
        ],
        out_specs=pl.BlockSpec((B, D), lambda i: (i, 0)),
        out_shape=jax.ShapeDtypeStruct((E, D), jnp.float32),
    )(edge_attr, W_e)


def _tc_final_body(h_ref, sc_ref, dis_ref, wg_ref, alpha_ref, eps_ref, o_ref):
    w = _softmax_w(alpha_ref)
    w1 = w[0:1, 1:2]
    h = h_ref[...]
    pre = h * (1.0 + eps_ref[...]) + sc_ref[1]
    gine = jnp.dot(pre, wg_ref[...], preferred_element_type=jnp.float32,
                   precision=lax.Precision.HIGHEST)
    o_ref[...] = dis_ref[...] * sc_ref[0] + gine * w1


def _tc_final(H, scout, dis, W_gine, alpha2, eps2):
    B = 2000
    return pl.pallas_call(
        _tc_final_body,
        grid=(N // B,),
        in_specs=[
            pl.BlockSpec((B, D), lambda i: (i, 0)),
            pl.BlockSpec((2, B, D), lambda i: (0, i, 0)),
            pl.BlockSpec((B, 1), lambda i: (i, 0)),
            pl.BlockSpec((D, D), lambda i: (0, 0)),
            pl.BlockSpec((1, 2), lambda i: (0, 0)),
            pl.BlockSpec((1, 1), lambda i: (0, 0)),
        ],
        out_specs=pl.BlockSpec((B, D), lambda i: (i, 0)),
        out_shape=jax.ShapeDtypeStruct((N, D), jnp.float32),
    )(H, scout, dis, W_gine, alpha2, eps2)


def kernel(H, edge_index, edge_attr, alpha, W_gcn, W_e, W_gine, eps):
    ei_pad = jnp.pad(edge_index, ((0, 0), (0, NCHUNK_PAD * CHUNK - E)))
    src2d = ei_pad[0].reshape(NCHUNK_PAD, 1, CHUNK)
    dst2d = ei_pad[1].reshape(NCHUNK_PAD, 1, CHUNK)
    alpha2 = alpha.reshape(1, 2)
    eps2 = eps.reshape(1, 1)

    degp = _sc_degree(dst2d)
    etab = _tc_etab(edge_attr, W_e)
    xw2, dis = _tc_prep(H, W_gcn, degp, alpha2)
    scout = _sc_main(xw2, H, etab, src2d, dst2d)
    return _tc_final(H, scout, dis, W_gine, alpha2, eps2)


# speedup vs baseline: 9.5672x; 1.0020x over previous
import functools

import jax
import jax.numpy as jnp
from jax import lax
from jax.experimental import pallas as pl
from jax.experimental.pallas import tpu as pltpu
from jax.experimental.pallas import tpu_sc as plsc

N = 10000
E = 320000
D = 128
DE = 16

CHUNK = 128
NCHUNK = E // CHUNK
NCHUNK_PAD = 2512
NTILE = 16
NPAD = 10240
ROWS_PER_TILE = NPAD // NTILE

RING = 16
NWIN = 10
EQ = 32
DEG_WIN = 79

_MESH = plsc.VectorSubcoreMesh(core_axis_name="c", subcore_axis_name="s")


@functools.partial(
    pl.kernel,
    out_type=jax.ShapeDtypeStruct((2, NPAD, D), jnp.float32),
    mesh=_MESH,
    scratch_types=[
        pltpu.VMEM((CHUNK, D), jnp.float32),
        pltpu.VMEM((DEG_WIN, 1, CHUNK), jnp.int32),
        pltpu.VMEM((CHUNK, D), jnp.float32),
        pltpu.VMEM_SHARED((NPAD, D), jnp.float32),
        pltpu.SemaphoreType.DMA,
    ],
)
def _sc_degree(dst_hbm, out_hbm, zb_v, idx_v, ones_v, acc_sh, dsem):
    c = lax.axis_index("c")
    s = lax.axis_index("s")

    @pl.loop(0, CHUNK)
    def _(r):
        for k in range(D // 16):
            zb_v[r, pl.ds(16 * k, 16)] = jnp.zeros((16,), jnp.float32)
            ones_v[r, pl.ds(16 * k, 16)] = jnp.ones((16,), jnp.float32)

    start = c * (NCHUNK // 2) + s * 78 + jnp.minimum(s, 2)
    cnt = jnp.where(s < 2, 79, 78)
    pltpu.sync_copy(dst_hbm.at[pl.ds(start, DEG_WIN)], idx_v)

    for q in range(5):
        pltpu.sync_copy(
            zb_v, acc_sh.at[pl.ds(s * ROWS_PER_TILE + q * CHUNK, CHUNK)])
    plsc.subcore_barrier()

    @pl.loop(0, DEG_WIN)
    def _(i):
        @pl.when(i < cnt)
        def _():
            pltpu.make_async_copy(
                ones_v, acc_sh.at[idx_v.at[i].at[0]], dsem).start(add=True)

    @pl.loop(0, DEG_WIN)
    def _(i):
        @pl.when(i < cnt)
        def _():
            pltpu.make_async_copy(
                ones_v, acc_sh.at[idx_v.at[i].at[0]], dsem).wait()

    plsc.subcore_barrier()
    for q in range(5):
        pltpu.sync_copy(
            acc_sh.at[pl.ds(s * ROWS_PER_TILE + q * CHUNK, CHUNK)],
            out_hbm.at[c].at[pl.ds(s * ROWS_PER_TILE + q * CHUNK, CHUNK)],
        )


@functools.partial(
    pl.kernel,
    out_type=jax.ShapeDtypeStruct((2, NPAD, D), jnp.float32),
    mesh=_MESH,
    scratch_types=[
        pltpu.VMEM((RING, 1, CHUNK), jnp.int32),
        pltpu.VMEM((RING, 1, CHUNK), jnp.int32),
        pltpu.VMEM((CHUNK, D), jnp.float32),
        pltpu.VMEM((CHUNK, D), jnp.float32),
        pltpu.VMEM_SHARED((NPAD, D), jnp.float32),
        pltpu.SemaphoreType.DMA,
        pltpu.SemaphoreType.DMA,
        pltpu.SemaphoreType.DMA,
        pltpu.SemaphoreType.DMA,
    ],
)
def _sc_main(xw2_hbm, h_hbm, e_hbm, src_hbm, dst_hbm, out_hbm,
             srcv, dstv, rows0, rows1, acc_sh,
             semg0, semg1, seme0, seme1):
    c = lax.axis_index("c")
    s = lax.axis_index("s")
    rows = (rows0, rows1)
    semg = (semg0, semg1)
    seme = (seme0, seme1)

    start = s * 156 + jnp.minimum(s, 4)
    cnt = jnp.where(s < 4, 157, 156)

    @pl.loop(0, CHUNK)
    def _(r):
        for k in range(D // 16):
            rows0[r, pl.ds(16 * k, 16)] = jnp.zeros((16,), jnp.float32)

    for q in range(5):
        pltpu.sync_copy(
            rows0, acc_sh.at[pl.ds(s * ROWS_PER_TILE + q * CHUNK, CHUNK)])
    plsc.subcore_barrier()

    def gcn_window(w):
        pltpu.sync_copy(src_hbm.at[pl.ds(start + w * RING, RING)], srcv)
        pltpu.sync_copy(dst_hbm.at[pl.ds(start + w * RING, RING)], dstv)

        for par in (0, 1):
            i = w * RING + par

            @pl.when(i < cnt)
            def _():
                pltpu.make_async_copy(
                    xw2_hbm.at[srcv.at[par].at[0]], rows[par],
                    semg[par]).start()

        @pl.loop(0, RING // 2)
        def _(t):
            for par in (0, 1):
                r = 2 * t + par
                i = w * RING + r

                @pl.when(i < cnt)
                def _():
                    pltpu.make_async_copy(
                        xw2_hbm.at[srcv.at[r].at[0]], rows[par],
                        semg[par]).wait()
                    pltpu.sync_copy(
                        rows[par], acc_sh.at[dstv.at[r].at[0]], add=True)

                    @pl.when((r + 2 < RING) & (i + 2 < cnt))
                    def _():
                        pltpu.make_async_copy(
                            xw2_hbm.at[srcv.at[r + 2].at[0]], rows[par],
                            semg[par]).start()

    def gine_window(w):
        pltpu.sync_copy(src_hbm.at[pl.ds(start + w * RING, RING)], srcv)
        pltpu.sync_copy(dst_hbm.at[pl.ds(start + w * RING, RING)], dstv)

        @pl.when(w * RING < cnt)
        def _():
            pltpu.make_async_copy(
                e_hbm.at[pl.ds((start + w * RING) * CHUNK, CHUNK)],
                rows[0], seme[0]).wait()
            pltpu.make_async_copy(
                h_hbm.at[srcv.at[0].at[0]], rows[0], semg[0]
            ).start(add=True)

        @pl.loop(0, RING // 2)
        def _(t):
            for par in (0, 1):
                r = 2 * t + par
                i = w * RING + r

                @pl.when(i < cnt)
                def _():
                    pltpu.make_async_copy(
                        h_hbm.at[srcv.at[r].at[0]], rows[par],
                        semg[par]).wait()

                    @pl.when((r + 1 < RING) & (i + 1 < cnt))
                    def _():
                        pltpu.make_async_copy(
                            e_hbm.at[pl.ds((start + i + 1) * CHUNK, CHUNK)],
                            rows[1 - par], seme[1 - par]).wait()
                        pltpu.make_async_copy(
                            h_hbm.at[srcv.at[r + 1].at[0]], rows[1 - par],
                            semg[1 - par]).start(add=True)

                    @pl.loop(0, CHUNK)
                    def _(rr):
                        for k in range(D // 16):
                            a = rows[par][rr, pl.ds(16 * k, 16)]
                            rows[par][rr, pl.ds(16 * k, 16)] = jnp.maximum(
                                a, 0.0)

                    pltpu.sync_copy(
                        rows[par], acc_sh.at[dstv.at[r].at[0]], add=True)

                    @pl.when(i + 2 < cnt)
                    def _():
                        pltpu.make_async_copy(
                            e_hbm.at[pl.ds((start + i + 2) * CHUNK, CHUNK)],
                            rows[par], seme[par]).start()

    @pl.when(c == 0)
    def _():
        @pl.loop(0, NWIN)
        def _(w):
            gcn_window(w)

    @pl.when(c == 1)
    def _():
        for par in (0, 1):
            pltpu.make_async_copy(
                e_hbm.at[pl.ds((start + par) * CHUNK, CHUNK)], rows[par],
                seme[par]).start()

        @pl.loop(0, NWIN)
        def _(w):
            gine_window(w)

    plsc.subcore_barrier()
    for q in range(5):
        pltpu.sync_copy(
            acc_sh.at[pl.ds(s * ROWS_PER_TILE + q * CHUNK, CHUNK)],
            out_hbm.at[c].at[pl.ds(s * ROWS_PER_TILE + q * CHUNK, CHUNK)],
        )


def _softmax_w(alpha_ref):
    al = alpha_ref[...]
    ex = jnp.exp(al - jnp.max(al))
    return ex / jnp.sum(ex)


def _tc_prep_body(h_ref, w_ref, degp_ref, alpha_ref, xw2_ref, dis_ref):
    w = _softmax_w(alpha_ref)
    w0 = w[0:1, 0:1]
    deg = degp_ref[0, :, 0:1] + degp_ref[1, :, 0:1]
    dis = jnp.where(deg > 0, lax.rsqrt(jnp.maximum(deg, 1.0)), 0.0)
    xw = jnp.dot(h_ref[...], w_ref[...], preferred_element_type=jnp.float32,
                 precision=lax.Precision.HIGHEST)
    xw2_ref[...] = xw * (dis * w0)
    dis_ref[...] = dis


def _tc_prep(H, W_gcn, degp, alpha2):
    B = 2000
    return pl.pallas_call(
        _tc_prep_body,
        grid=(N // B,),
        in_specs=[
            pl.BlockSpec((B, D), lambda i: (i, 0)),
            pl.BlockSpec((D, D), lambda i: (0, 0)),
            pl.BlockSpec((2, B, D), lambda i: (0, i, 0)),
            pl.BlockSpec((1, 2), lambda i: (0, 0)),
        ],
        out_specs=[
            pl.BlockSpec((B, D), lambda i: (i, 0)),
            pl.BlockSpec((B, 1), lambda i: (i, 0)),
        ],
        out_shape=[
            jax.ShapeDtypeStruct((N, D), jnp.float32),
            jax.ShapeDtypeStruct((N, 1), jnp.float32),
        ],
    )(H, W_gcn, degp, alpha2)


def _tc_etab_body(ea_ref, we_ref, e_ref):
    e_ref[...] = jnp.dot(ea_ref[...], we_ref[...],
                         preferred_element_type=jnp.float32,
                         precision=lax.Precision.HIGHEST)


def _tc_etab(edge_attr, W_e):
    B = 3200
    return pl.pallas_call(
        _tc_etab_body,
        grid=(E // B,),
        in_specs=[
            pl.BlockSpec((B, DE), lambda i: (i, 0)),
            pl.BlockSpec((DE, D), lambda i: (0, 0)),
        ],
        out_specs=pl.BlockSpec((B, D), lambda i: (i, 0)),
        out_shape=jax.ShapeDtypeStruct((E, D), jnp.float32),
    )(edge_attr, W_e)


def _tc_final_body(h_ref, sc_ref, dis_ref, wg_ref, alpha_ref, eps_ref, o_ref):
    w = _softmax_w(alpha_ref)
    w1 = w[0:1, 1:2]
    h = h_ref[...]
    pre = h * (1.0 + eps_ref[...]) + sc_ref[1]
    gine = jnp.dot(pre, wg_ref[...], preferred_element_type=jnp.float32,
                   precision=lax.Precision.HIGHEST)
    o_ref[...] = dis_ref[...] * sc_ref[0] + gine * w1


def _tc_final(H, scout, dis, W_gine, alpha2, eps2):
    B = 2000
    return pl.pallas_call(
        _tc_final_body,
        grid=(N // B,),
        in_specs=[
            pl.BlockSpec((B, D), lambda i: (i, 0)),
            pl.BlockSpec((2, B, D), lambda i: (0, i, 0)),
            pl.BlockSpec((B, 1), lambda i: (i, 0)),
            pl.BlockSpec((D, D), lambda i: (0, 0)),
            pl.BlockSpec((1, 2), lambda i: (0, 0)),
            pl.BlockSpec((1, 1), lambda i: (0, 0)),
        ],
        out_specs=pl.BlockSpec((B, D), lambda i: (i, 0)),
        out_shape=jax.ShapeDtypeStruct((N, D), jnp.float32),
    )(H, scout, dis, W_gine, alpha2, eps2)


def kernel(H, edge_index, edge_attr, alpha, W_gcn, W_e, W_gine, eps):
    ei_pad = jnp.pad(edge_index, ((0, 0), (0, NCHUNK_PAD * CHUNK - E)))
    src2d = ei_pad[0].reshape(NCHUNK_PAD, 1, CHUNK)
    dst2d = ei_pad[1].reshape(NCHUNK_PAD, 1, CHUNK)
    alpha2 = alpha.reshape(1, 2)
    eps2 = eps.reshape(1, 1)

    degp = _sc_degree(dst2d)
    etab = _tc_etab(edge_attr, W_e)
    xw2, dis = _tc_prep(H, W_gcn, degp, alpha2)
    scout = _sc_main(xw2, H, etab, src2d, dst2d)
    return _tc_final(H, scout, dis, W_gine, alpha2, eps2)
